# baseline (device time: 76098 ns/iter reference)
import functools

import jax
import jax.numpy as jnp
from jax import lax
from jax.experimental import pallas as pl
from jax.experimental.pallas import tpu as pltpu

N_HEADS = 16
DH = 128
DR = 32
SCALE = (DH + DR) ** -0.5
N_O_CHUNKS = 4
N_W_CHUNKS = 4

_sem_signal = getattr(pl, "semaphore_signal", None) or pltpu.semaphore_signal
_sem_wait = getattr(pl, "semaphore_wait", None) or pltpu.semaphore_wait
_DeviceIdType = getattr(pl, "DeviceIdType", None) or pltpu.DeviceIdType
_CompilerParams = getattr(pltpu, "CompilerParams", None) or pltpu.TPUCompilerParams


def _mm(a, b):
    return lax.dot_general(
        a, b, (((1,), (0,)), ((), ())), preferred_element_type=jnp.float32
    )


def _mm_t(a, b):
    return lax.dot_general(
        a, b, (((1,), (1,)), ((), ())), preferred_element_type=jnp.float32
    )


def kernel(x, Wdkv, Wuk, Wuv, Wq, Wqr, Wkr, Wo):
    bf16 = jnp.bfloat16

    B, S, D = x.shape
    Dc_loc = Wdkv.shape[1]
    S_loc = S // 2
    WCW = D // N_W_CHUNKS

    def body(
        x_ref,
        wdkv_ref,
        wuk_ref,
        wuv_ref,
        wq_ref,
        wqr_ref,
        wkr_ref,
        wo_ref,
        out_ref,
        wuk_full,
        wuv_full,
        c_full,
        x_bf,
        q_buf,
        qr_buf,
        kr_buf,
        k_buf,
        v_buf,
        o_mine,
        o_peer,
        w_stage,
        y_send_sems,
        y_recv_sems,
        x_send_sems,
        x_recv_sems,
        w_sems,
    ):
        my_x = lax.axis_index("x")
        my_y = lax.axis_index("y")
        y_peer = (my_x, 1 - my_y)
        x_peer = (1 - my_x, my_y)

        def stream_weight(w_ref, consume):
            cps = []
            cp0 = pltpu.make_async_copy(
                w_ref.at[:, pl.ds(0, WCW)], w_stage.at[0], w_sems.at[0]
            )
            cp0.start()
            cps.append(cp0)
            for j in range(N_W_CHUNKS):
                if j + 1 < N_W_CHUNKS:
                    nxt = pltpu.make_async_copy(
                        w_ref.at[:, pl.ds((j + 1) * WCW, WCW)],
                        w_stage.at[(j + 1) % 2],
                        w_sems.at[(j + 1) % 2],
                    )
                    nxt.start()
                    cps.append(nxt)
                cps[j].wait()
                consume(j, w_stage[j % 2].astype(bf16))

        barrier = pltpu.get_barrier_semaphore()
        _sem_signal(barrier, inc=1, device_id=y_peer, device_id_type=_DeviceIdType.MESH)
        _sem_signal(barrier, inc=1, device_id=x_peer, device_id_type=_DeviceIdType.MESH)
        _sem_wait(barrier, 2)

        woff = my_y * Dc_loc
        wuk_full[pl.ds(woff, Dc_loc), :] = wuk_ref[...].astype(bf16)
        rdma_wuk = pltpu.make_async_remote_copy(
            src_ref=wuk_full.at[pl.ds(woff, Dc_loc), :],
            dst_ref=wuk_full.at[pl.ds(woff, Dc_loc), :],
            send_sem=y_send_sems.at[0],
            recv_sem=y_recv_sems.at[0],
            device_id=y_peer,
            device_id_type=_DeviceIdType.MESH,
        )
        rdma_wuk.start()
        wuv_full[pl.ds(woff, Dc_loc), :] = wuv_ref[...].astype(bf16)
        rdma_wuv = pltpu.make_async_remote_copy(
            src_ref=wuv_full.at[pl.ds(woff, Dc_loc), :],
            dst_ref=wuv_full.at[pl.ds(woff, Dc_loc), :],
            send_sem=y_send_sems.at[1],
            recv_sem=y_recv_sems.at[1],
            device_id=y_peer,
            device_id_type=_DeviceIdType.MESH,
        )
        rdma_wuv.start()

        x_bf[...] = x_ref[0].astype(bf16)

        c_full[:, pl.ds(woff, Dc_loc)] = _mm(
            x_bf[...], wdkv_ref[...].astype(bf16)
        ).astype(bf16)
        rdma_c = pltpu.make_async_remote_copy(
            src_ref=c_full.at[:, pl.ds(woff, Dc_loc)],
            dst_ref=c_full.at[:, pl.ds(woff, Dc_loc)],
            send_sem=y_send_sems.at[2],
            recv_sem=y_recv_sems.at[2],
            device_id=y_peer,
            device_id_type=_DeviceIdType.MESH,
        )
        rdma_c.start()

        row0 = my_x * S_loc
        xq = x_bf[pl.ds(row0, S_loc), :]

        def q_chunk(j, wq_bf):
            q_buf[:, j * WCW : (j + 1) * WCW] = (_mm(xq, wq_bf) * SCALE).astype(bf16)

        stream_weight(wq_ref, q_chunk)
        qr_buf[...] = (_mm(xq, wqr_ref[...].astype(bf16)) * SCALE).astype(bf16)
        kr_buf[...] = _mm(x_bf[...], wkr_ref[...].astype(bf16)).astype(bf16)

        rdma_wuk.wait()
        rdma_wuv.wait()
        rdma_c.wait()

        k_buf[...] = _mm(c_full[...], wuk_full[...]).astype(bf16)
        v_buf[...] = _mm(c_full[...], wuv_full[...]).astype(bf16)

        HPC = N_HEADS // N_O_CHUNKS
        CW = HPC * DH
        rdma_o = []
        for h in range(N_HEADS):
            q = q_buf[:, h * DH : (h + 1) * DH]
            k = k_buf[:, h * DH : (h + 1) * DH]
            qr = qr_buf[:, h * DR : (h + 1) * DR]
            s = _mm_t(q, k) + _mm_t(qr, kr_buf[...])
            p = jnp.exp(s)
            denom = jnp.sum(p, axis=-1, keepdims=True)
            o = _mm(p.astype(bf16), v_buf[:, h * DH : (h + 1) * DH])
            o_mine[:, h * DH : (h + 1) * DH] = (o / denom).astype(bf16)
            if (h + 1) % HPC == 0:
                i = h // HPC
                rdma = pltpu.make_async_remote_copy(
                    src_ref=o_mine.at[:, pl.ds(i * CW, CW)],
                    dst_ref=o_peer.at[:, pl.ds(i * CW, CW)],
                    send_sem=x_send_sems.at[i],
                    recv_sem=x_recv_sems.at[i],
                    device_id=x_peer,
                    device_id_type=_DeviceIdType.MESH,
                )
                rdma.start()
                rdma_o.append(rdma)

        peer_row0 = (1 - my_x) * S_loc

        def out_mine_chunk(j, wo_bf):
            out_ref[0, pl.ds(row0, S_loc), pl.ds(j * WCW, WCW)] = _mm(
                o_mine[...], wo_bf
            )

        stream_weight(wo_ref, out_mine_chunk)
        for rdma in rdma_o:
            rdma.wait_recv()

        def out_peer_chunk(j, wo_bf):
            out_ref[0, pl.ds(peer_row0, S_loc), pl.ds(j * WCW, WCW)] = _mm(
                o_peer[...], wo_bf
            )

        stream_weight(wo_ref, out_peer_chunk)
        for rdma in rdma_o:
            rdma.wait_send()

        @functools.partial(pl.run_scoped, sem=pltpu.SemaphoreType.REGULAR)
        def _(sem):
            _sem_signal(sem, inc=1, device_id=y_peer, device_id_type=_DeviceIdType.MESH)
            _sem_signal(sem, inc=1, device_id=x_peer, device_id_type=_DeviceIdType.MESH)
            _sem_wait(sem, 2)

    out_shape = jax.ShapeDtypeStruct((B, S, D), jnp.float32)
    vmem = pl.BlockSpec(memory_space=pltpu.VMEM)
    hbm = pl.BlockSpec(memory_space=pl.ANY)
    return pl.pallas_call(
        body,
        out_shape=out_shape,
        in_specs=[vmem, vmem, vmem, vmem, hbm, vmem, vmem, hbm],
        out_specs=vmem,
        scratch_shapes=[
            pltpu.VMEM((2 * Dc_loc, D), bf16),
            pltpu.VMEM((2 * Dc_loc, D), bf16),
            pltpu.VMEM((S, 2 * Dc_loc), bf16),
            pltpu.VMEM((S, D), bf16),
            pltpu.VMEM((S_loc, N_HEADS * DH), bf16),
            pltpu.VMEM((S_loc, N_HEADS * DR), bf16),
            pltpu.VMEM((S, DR), bf16),
            pltpu.VMEM((S, N_HEADS * DH), bf16),
            pltpu.VMEM((S, N_HEADS * DH), bf16),
            pltpu.VMEM((S_loc, N_HEADS * DH), bf16),
            pltpu.VMEM((S_loc, N_HEADS * DH), bf16),
            pltpu.VMEM((2, D, WCW), jnp.float32),
            pltpu.SemaphoreType.DMA((3,)),
            pltpu.SemaphoreType.DMA((3,)),
            pltpu.SemaphoreType.DMA((N_O_CHUNKS,)),
            pltpu.SemaphoreType.DMA((N_O_CHUNKS,)),
            pltpu.SemaphoreType.DMA((2,)),
        ],
        compiler_params=_CompilerParams(
            collective_id=0, vmem_limit_bytes=66_900_000
        ),
    )(x, Wdkv, Wuk, Wuv, Wq, Wqr, Wkr, Wo)


# device time: 74038 ns/iter; 1.0278x vs baseline; 1.0278x over previous
import functools

import jax
import jax.numpy as jnp
from jax import lax
from jax.experimental import pallas as pl
from jax.experimental.pallas import tpu as pltpu

N_HEADS = 16
DH = 128
DR = 32
SCALE = (DH + DR) ** -0.5
N_O_CHUNKS = 8
N_W_CHUNKS = 4

_sem_signal = getattr(pl, "semaphore_signal", None) or pltpu.semaphore_signal
_sem_wait = getattr(pl, "semaphore_wait", None) or pltpu.semaphore_wait
_DeviceIdType = getattr(pl, "DeviceIdType", None) or pltpu.DeviceIdType
_CompilerParams = getattr(pltpu, "CompilerParams", None) or pltpu.TPUCompilerParams


def _mm(a, b):
    return lax.dot_general(
        a, b, (((1,), (0,)), ((), ())), preferred_element_type=jnp.float32
    )


def _mm_t(a, b):
    return lax.dot_general(
        a, b, (((1,), (1,)), ((), ())), preferred_element_type=jnp.float32
    )


def kernel(x, Wdkv, Wuk, Wuv, Wq, Wqr, Wkr, Wo):
    bf16 = jnp.bfloat16

    B, S, D = x.shape
    Dc_loc = Wdkv.shape[1]
    S_loc = S // 2
    WCW = D // N_W_CHUNKS

    def body(
        x_ref,
        wdkv_ref,
        wuk_ref,
        wuv_ref,
        wq_ref,
        wqr_ref,
        wkr_ref,
        wo_ref,
        out_ref,
        wuk_full,
        wuv_full,
        c_full,
        x_bf,
        q_buf,
        qr_buf,
        kr_buf,
        k_buf,
        v_buf,
        o_mine,
        o_peer,
        w_stage,
        y_send_sems,
        y_recv_sems,
        x_send_sems,
        x_recv_sems,
        w_sems,
    ):
        my_x = lax.axis_index("x")
        my_y = lax.axis_index("y")
        y_peer = (my_x, 1 - my_y)
        x_peer = (1 - my_x, my_y)

        def stream_weight(w_ref, consume):
            cps = []
            cp0 = pltpu.make_async_copy(
                w_ref.at[:, pl.ds(0, WCW)], w_stage.at[0], w_sems.at[0]
            )
            cp0.start()
            cps.append(cp0)
            for j in range(N_W_CHUNKS):
                if j + 1 < N_W_CHUNKS:
                    nxt = pltpu.make_async_copy(
                        w_ref.at[:, pl.ds((j + 1) * WCW, WCW)],
                        w_stage.at[(j + 1) % 2],
                        w_sems.at[(j + 1) % 2],
                    )
                    nxt.start()
                    cps.append(nxt)
                cps[j].wait()
                consume(j, w_stage[j % 2].astype(bf16))

        barrier = pltpu.get_barrier_semaphore()
        _sem_signal(barrier, inc=1, device_id=y_peer, device_id_type=_DeviceIdType.MESH)
        _sem_signal(barrier, inc=1, device_id=x_peer, device_id_type=_DeviceIdType.MESH)
        _sem_wait(barrier, 2)

        woff = my_y * Dc_loc
        wuk_full[pl.ds(woff, Dc_loc), :] = wuk_ref[...].astype(bf16)
        rdma_wuk = pltpu.make_async_remote_copy(
            src_ref=wuk_full.at[pl.ds(woff, Dc_loc), :],
            dst_ref=wuk_full.at[pl.ds(woff, Dc_loc), :],
            send_sem=y_send_sems.at[0],
            recv_sem=y_recv_sems.at[0],
            device_id=y_peer,
            device_id_type=_DeviceIdType.MESH,
        )
        rdma_wuk.start()
        wuv_full[pl.ds(woff, Dc_loc), :] = wuv_ref[...].astype(bf16)
        rdma_wuv = pltpu.make_async_remote_copy(
            src_ref=wuv_full.at[pl.ds(woff, Dc_loc), :],
            dst_ref=wuv_full.at[pl.ds(woff, Dc_loc), :],
            send_sem=y_send_sems.at[1],
            recv_sem=y_recv_sems.at[1],
            device_id=y_peer,
            device_id_type=_DeviceIdType.MESH,
        )
        rdma_wuv.start()

        x_bf[...] = x_ref[0].astype(bf16)

        c_full[:, pl.ds(woff, Dc_loc)] = _mm(
            x_bf[...], wdkv_ref[...].astype(bf16)
        ).astype(bf16)
        rdma_c = pltpu.make_async_remote_copy(
            src_ref=c_full.at[:, pl.ds(woff, Dc_loc)],
            dst_ref=c_full.at[:, pl.ds(woff, Dc_loc)],
            send_sem=y_send_sems.at[2],
            recv_sem=y_recv_sems.at[2],
            device_id=y_peer,
            device_id_type=_DeviceIdType.MESH,
        )
        rdma_c.start()

        row0 = my_x * S_loc
        xq = x_bf[pl.ds(row0, S_loc), :]

        def q_chunk(j, wq_bf):
            q_buf[:, j * WCW : (j + 1) * WCW] = (_mm(xq, wq_bf) * SCALE).astype(bf16)

        stream_weight(wq_ref, q_chunk)
        qr_buf[...] = (_mm(xq, wqr_ref[...].astype(bf16)) * SCALE).astype(bf16)
        kr_buf[...] = _mm(x_bf[...], wkr_ref[...].astype(bf16)).astype(bf16)

        rdma_wuk.wait()
        rdma_wuv.wait()
        rdma_c.wait()

        k_buf[...] = _mm(c_full[...], wuk_full[...]).astype(bf16)
        v_buf[...] = _mm(c_full[...], wuv_full[...]).astype(bf16)

        HPC = N_HEADS // N_O_CHUNKS
        CW = HPC * DH
        rdma_o = []
        for h in range(N_HEADS):
            q = q_buf[:, h * DH : (h + 1) * DH]
            k = k_buf[:, h * DH : (h + 1) * DH]
            qr = qr_buf[:, h * DR : (h + 1) * DR]
            s = _mm_t(q, k) + _mm_t(qr, kr_buf[...])
            p = jnp.exp(s)
            denom = jnp.sum(p, axis=-1, keepdims=True)
            o = _mm(p.astype(bf16), v_buf[:, h * DH : (h + 1) * DH])
            o_mine[:, h * DH : (h + 1) * DH] = (o / denom).astype(bf16)
            if (h + 1) % HPC == 0:
                i = h // HPC
                rdma = pltpu.make_async_remote_copy(
                    src_ref=o_mine.at[:, pl.ds(i * CW, CW)],
                    dst_ref=o_peer.at[:, pl.ds(i * CW, CW)],
                    send_sem=x_send_sems.at[i],
                    recv_sem=x_recv_sems.at[i],
                    device_id=x_peer,
                    device_id_type=_DeviceIdType.MESH,
                )
                rdma.start()
                rdma_o.append(rdma)

        peer_row0 = (1 - my_x) * S_loc

        def out_mine_chunk(j, wo_bf):
            out_ref[0, pl.ds(row0, S_loc), pl.ds(j * WCW, WCW)] = _mm(
                o_mine[...], wo_bf
            )

        stream_weight(wo_ref, out_mine_chunk)
        for rdma in rdma_o:
            rdma.wait_recv()

        def out_peer_chunk(j, wo_bf):
            out_ref[0, pl.ds(peer_row0, S_loc), pl.ds(j * WCW, WCW)] = _mm(
                o_peer[...], wo_bf
            )

        stream_weight(wo_ref, out_peer_chunk)
        for rdma in rdma_o:
            rdma.wait_send()

        @functools.partial(pl.run_scoped, sem=pltpu.SemaphoreType.REGULAR)
        def _(sem):
            _sem_signal(sem, inc=1, device_id=y_peer, device_id_type=_DeviceIdType.MESH)
            _sem_signal(sem, inc=1, device_id=x_peer, device_id_type=_DeviceIdType.MESH)
            _sem_wait(sem, 2)

    out_shape = jax.ShapeDtypeStruct((B, S, D), jnp.float32)
    vmem = pl.BlockSpec(memory_space=pltpu.VMEM)
    hbm = pl.BlockSpec(memory_space=pl.ANY)
    return pl.pallas_call(
        body,
        out_shape=out_shape,
        in_specs=[vmem, vmem, vmem, vmem, hbm, vmem, vmem, hbm],
        out_specs=vmem,
        scratch_shapes=[
            pltpu.VMEM((2 * Dc_loc, D), bf16),
            pltpu.VMEM((2 * Dc_loc, D), bf16),
            pltpu.VMEM((S, 2 * Dc_loc), bf16),
            pltpu.VMEM((S, D), bf16),
            pltpu.VMEM((S_loc, N_HEADS * DH), bf16),
            pltpu.VMEM((S_loc, N_HEADS * DR), bf16),
            pltpu.VMEM((S, DR), bf16),
            pltpu.VMEM((S, N_HEADS * DH), bf16),
            pltpu.VMEM((S, N_HEADS * DH), bf16),
            pltpu.VMEM((S_loc, N_HEADS * DH), bf16),
            pltpu.VMEM((S_loc, N_HEADS * DH), bf16),
            pltpu.VMEM((2, D, WCW), jnp.float32),
            pltpu.SemaphoreType.DMA((3,)),
            pltpu.SemaphoreType.DMA((3,)),
            pltpu.SemaphoreType.DMA((N_O_CHUNKS,)),
            pltpu.SemaphoreType.DMA((N_O_CHUNKS,)),
            pltpu.SemaphoreType.DMA((2,)),
        ],
        compiler_params=_CompilerParams(
            collective_id=0, vmem_limit_bytes=66_900_000
        ),
    )(x, Wdkv, Wuk, Wuv, Wq, Wqr, Wkr, Wo)
